# baseline (speedup 1.0000x reference)
import functools

import jax
import jax.numpy as jnp
from jax import lax
from jax.experimental import pallas as pl
from jax.experimental.pallas import tpu as pltpu
from jax.experimental.pallas import tpu_sc as plsc

_PAD = 1
_BSZ = 4
_SEQ = 8192
_D = 1024
_CHUNK = 1024
_ROWS = 512
_G = 32
_NWIN = _ROWS // _G
_INV = 8208
_PSINK = 8200


def _body(inp_hbm, w_hbm, out_hbm, inp_v, idx_v, sidx2d, vals2d, pad2048,
          pad2d, tot_tmp, totals_v, dflat, d2d0, d2d1, ones16, tmp16,
          fixsrc, fixdst, buf0, buf1, buf2,
          shared_tot, inv0, inv1,
          gsem0, gsem1, gsem2, wsem0, wsem1, wsem2, isem):
    core = lax.axis_index("c")
    sid = lax.axis_index("s")
    bb = sid // 8
    b = core * 2 + bb
    chunk = sid % 8
    base = pl.multiple_of(chunk * _CHUNK, _CHUNK)

    pltpu.sync_copy(inp_hbm.at[b, pl.ds(base, _CHUNK)], inp_v)

    ones16[...] = jnp.full((16,), _PAD, jnp.int32)

    run = jnp.int32(0)
    for i in range(_CHUNK // 16):
        x = inp_v[pl.ds(i * 16, 16)]
        m = jnp.minimum(jnp.abs(x - _PAD), 1)
        cum = plsc.cumsum(m) + run
        idx_v[pl.ds(i * 16, 16)] = cum * m + _PAD
        run = run + jnp.sum(m)

    tot_tmp[...] = jnp.full((16,), run, jnp.int32)
    pltpu.sync_copy(tot_tmp, shared_tot.at[sid])
    plsc.subcore_barrier()
    pltpu.sync_copy(shared_tot, totals_v)
    row0 = bb * 8
    off = jnp.int32(0)
    tloc = [jnp.int32(0), jnp.int32(0)]
    for j in range(16):
        tj = totals_v[j][0]
        tloc[j // 8] = tloc[j // 8] + tj
        take = jnp.logical_and(j >= row0, j < sid)
        off = off + jnp.where(take, tj, jnp.int32(0))

    prun = jnp.int32(0)
    for i in range(_CHUNK // 16):
        v = idx_v[pl.ds(i * 16, 16)]
        mm = jnp.minimum(jnp.abs(v - _PAD), 1)
        pos = v + off * mm
        sidx = pos * mm + (1 - mm) * _PSINK
        sidx2d[i // 8, pl.ds((i % 8) * 16, 16)] = sidx
        gval = b * _SEQ + base + i * 16 + lax.iota(jnp.int32, 16)
        vals2d[i // 8, pl.ds((i % 8) * 16, 16)] = gval
        pm = 1 - mm
        prank = plsc.cumsum(pm) + prun
        pidx = pm * (prank - 1) + mm * (1024 + i * 16 + lax.iota(jnp.int32, 16))
        plsc.store_scatter(pad2048, [pidx], gval)
        prun = prun + jnp.sum(pm)
    npad = prun

    @pl.when(bb == 0)
    def _():
        hs = [pltpu.async_copy(vals2d.at[w2], inv0.at[sidx2d.at[w2]], isem)
              for w2 in range(8)]
        for h in hs:
            h.wait()

    @pl.when(bb == 1)
    def _():
        hs = [pltpu.async_copy(vals2d.at[w2], inv1.at[sidx2d.at[w2]], isem)
              for w2 in range(8)]
        for h in hs:
            h.wait()

    sink0 = pad2048[pl.ds(0, 16)][0]
    for i in range(64):
        v = pad2048[pl.ds(i * 16, 16)]
        kvec = lax.iota(jnp.int32, 16) + i * 16
        pad2d[i, pl.ds(0, 16)] = jnp.where(kvec < npad, v, sink0)

    plsc.subcore_barrier()

    R0 = pl.multiple_of(sid * _ROWS, 8)
    for q, (invq, d2dq) in enumerate(((inv0, d2d0), (inv1, d2d1))):
        tq = tloc[q]
        pltpu.sync_copy(invq.at[pl.ds(R0, _ROWS)], dflat)
        pltpu.sync_copy(invq.at[pl.ds(8192, 16)], tmp16)
        tail = tmp16[pl.ds(0, 16)]
        sinkq = jnp.where(tq < _SEQ, tail[8], tail[1])
        for i in range(_ROWS // 16):
            v = dflat[pl.ds(i * 16, 16)]
            krow = lax.iota(jnp.int32, 16) + i * 16 + R0
            good = jnp.logical_and(krow >= 2, krow <= tq + 1)
            d2dq[i // 2, pl.ds((i % 2) * 16, 16)] = jnp.where(good, v, sinkq)

    bufs = (buf0, buf1, buf2)
    gsems = (gsem0, gsem1, gsem2)
    wsems = (wsem0, wsem1, wsem2)

    def fire_read(w):
        return pltpu.async_copy(
            w_hbm.at[pl.ds(R0 + w * _G, _G), :], bufs[w % 3], gsems[w % 3])

    def fire_scat(w, d2dq):
        return pltpu.async_copy(
            bufs[w % 3], out_hbm.at[d2dq.at[w]], wsems[w % 3])

    rh = [None] * _NWIN
    sh = [None] * _NWIN
    rh[0] = fire_read(0)
    rh[1] = fire_read(1)
    for w in range(_NWIN):
        rh[w].wait()
        sh[w] = (fire_scat(w, d2d0), fire_scat(w, d2d1))
        h = w + 2
        if h < _NWIN:
            if w >= 1:
                sh[w - 1][0].wait()
                sh[w - 1][1].wait()
            rh[h] = fire_read(h)
    for w in range(_NWIN - 3, _NWIN):
        sh[w][0].wait()
        sh[w][1].wait()

    plsc.subcore_barrier()

    @pl.when(sid == 15)
    def _():
        t = lax.iota(jnp.int32, 16)
        for q, invq in ((0, inv0), (1, inv1)):
            tq = tloc[q]
            n_extra = jnp.clip(tq - 8190, 0, 2)
            pltpu.sync_copy(invq.at[pl.ds(8192, 16)], tmp16)
            lane = jnp.where(n_extra > 0,
                             jnp.minimum(t, n_extra - 1), jnp.int32(8))
            dest = plsc.load_gather(tmp16, [lane])
            src = jnp.where(n_extra > 0, 8192 + jnp.minimum(t, n_extra - 1),
                            jnp.int32(_PAD))
            fixsrc[q, pl.ds(0, 16)] = src
            fixdst[q, pl.ds(0, 16)] = dest
            pltpu.async_copy(w_hbm.at[fixsrc.at[q]],
                             buf0.at[pl.ds(0, 16), :], gsem0).wait()
            pltpu.async_copy(buf0.at[pl.ds(0, 16), :],
                             out_hbm.at[fixdst.at[q]], wsem0).wait()

    pltpu.async_copy(w_hbm.at[ones16], buf2.at[pl.ds(0, 16), :], gsem2).wait()

    def zbody(w3, carry):
        pltpu.sync_copy(buf2.at[pl.ds(0, 16), :], out_hbm.at[pad2d.at[w3]])
        return carry

    lax.fori_loop(0, (npad + 15) // 16, zbody, jnp.int32(0))


@jax.jit
def _sc_embed(inp, weights):
    mesh = plsc.VectorSubcoreMesh(core_axis_name="c", subcore_axis_name="s")
    run = functools.partial(
        pl.kernel,
        mesh=mesh,
        compiler_params=pltpu.CompilerParams(needs_layout_passes=False),
        out_type=jax.ShapeDtypeStruct((_BSZ * _SEQ, _D), jnp.float32),
        scratch_types=[
            pltpu.VMEM((_CHUNK,), jnp.int32),
            pltpu.VMEM((_CHUNK,), jnp.int32),
            pltpu.VMEM((8, 128), jnp.int32),
            pltpu.VMEM((8, 128), jnp.int32),
            pltpu.VMEM((2048,), jnp.int32),
            pltpu.VMEM((64, 16), jnp.int32),
            pltpu.VMEM((16,), jnp.int32),
            pltpu.VMEM((16, 16), jnp.int32),
            pltpu.VMEM((_ROWS,), jnp.int32),
            pltpu.VMEM((_NWIN, _G), jnp.int32),
            pltpu.VMEM((_NWIN, _G), jnp.int32),
            pltpu.VMEM((16,), jnp.int32),
            pltpu.VMEM((16,), jnp.int32),
            pltpu.VMEM((2, 16), jnp.int32),
            pltpu.VMEM((2, 16), jnp.int32),
            pltpu.VMEM((_G, _D), jnp.float32),
            pltpu.VMEM((_G, _D), jnp.float32),
            pltpu.VMEM((_G, _D), jnp.float32),
            pltpu.VMEM_SHARED((16, 16), jnp.int32),
            pltpu.VMEM_SHARED((_INV,), jnp.int32),
            pltpu.VMEM_SHARED((_INV,), jnp.int32),
            pltpu.SemaphoreType.DMA,
            pltpu.SemaphoreType.DMA,
            pltpu.SemaphoreType.DMA,
            pltpu.SemaphoreType.DMA,
            pltpu.SemaphoreType.DMA,
            pltpu.SemaphoreType.DMA,
            pltpu.SemaphoreType.DMA,
        ],
    )(_body)
    return run(inp, weights).reshape(_BSZ, _SEQ, _D)


def kernel(input, weights):
    return lax.stop_gradient(_sc_embed(input, weights))


# speedup vs baseline: 1.0618x; 1.0618x over previous
import functools

import jax
import jax.numpy as jnp
from jax import lax
from jax.experimental import pallas as pl
from jax.experimental.pallas import tpu as pltpu
from jax.experimental.pallas import tpu_sc as plsc

_PAD = 1
_BSZ = 4
_SEQ = 8192
_D = 1024
_NC = 2
_NS = 16
_CHUNK = _SEQ // (_NC * _NS // _BSZ)
_CPB = _NC * _NS // _BSZ
_G = 16
_NSTEP = _CHUNK // _G
_NBUF = 6


def _body(inp_hbm, w_hbm, out_hbm, inp_v, idx_v, tot_tmp, totals_v,
          buf0, buf1, buf2, buf3, buf4, buf5, shared_totals,
          gsem0, gsem1, gsem2, gsem3, gsem4, gsem5,
          wsem0, wsem1, wsem2, wsem3, wsem4, wsem5):
    core = lax.axis_index("c")
    sid = lax.axis_index("s")
    b = core * (_NS // _CPB) + sid // _CPB
    chunk = sid % _CPB
    base = pl.multiple_of(chunk * _CHUNK, _CHUNK)

    pltpu.sync_copy(inp_hbm.at[b, pl.ds(base, _CHUNK)], inp_v)

    run = jnp.int32(0)
    for i in range(_CHUNK // 16):
        x = inp_v[pl.ds(i * 16, 16)]
        m = jnp.minimum(jnp.abs(x - _PAD), 1)
        cum = plsc.cumsum(m) + run
        idx_v[pl.ds(i * 16, 16)] = cum * m + _PAD
        run = run + jnp.sum(m)

    tot_tmp[...] = jnp.full((16,), run, jnp.int32)
    pltpu.sync_copy(tot_tmp, shared_totals.at[sid])
    plsc.subcore_barrier()
    pltpu.sync_copy(shared_totals, totals_v)
    row_base = (sid // _CPB) * _CPB
    off = jnp.int32(0)
    for j in range(_NS):
        tj = totals_v[j][0]
        take = jnp.logical_and(j >= row_base, j < sid)
        off = off + jnp.where(take, tj, jnp.int32(0))

    for i in range(_CHUNK // 16):
        v = idx_v[pl.ds(i * 16, 16)]
        mm = jnp.minimum(jnp.abs(v - _PAD), 1)
        idx_v[pl.ds(i * 16, 16)] = v + off * mm

    bufs = (buf0, buf1, buf2, buf3, buf4, buf5)
    gsems = (gsem0, gsem1, gsem2, gsem3, gsem4, gsem5)
    wsems = (wsem0, wsem1, wsem2, wsem3, wsem4, wsem5)

    def fire_gather(g):
        return pltpu.async_copy(
            w_hbm.at[idx_v.at[pl.ds(g * _G, _G)]],
            bufs[g % _NBUF], gsems[g % _NBUF])

    def fire_write(g):
        row = b * _SEQ + base + g * _G
        return pltpu.async_copy(
            bufs[g % _NBUF],
            out_hbm.at[pl.ds(row, _G), :], wsems[g % _NBUF])

    gh = [None] * _NSTEP
    wh = [None] * _NSTEP
    for g in range(_NBUF - 1):
        gh[g] = fire_gather(g)
    for g in range(_NSTEP):
        gh[g].wait()
        wh[g] = fire_write(g)
        h = g + _NBUF - 1
        if h < _NSTEP:
            if g >= 1:
                wh[g - 1].wait()
            gh[h] = fire_gather(h)
    for g in range(_NSTEP - _NBUF, _NSTEP):
        wh[g].wait()


@jax.jit
def _sc_embed(inp, weights):
    mesh = plsc.VectorSubcoreMesh(core_axis_name="c", subcore_axis_name="s")
    run = functools.partial(
        pl.kernel,
        mesh=mesh,
        compiler_params=pltpu.CompilerParams(needs_layout_passes=False),
        out_type=jax.ShapeDtypeStruct((_BSZ * _SEQ, _D), jnp.float32),
        scratch_types=[
            pltpu.VMEM((_CHUNK,), jnp.int32),
            pltpu.VMEM((_CHUNK,), jnp.int32),
            pltpu.VMEM((16,), jnp.int32),
            pltpu.VMEM((_NS, 16), jnp.int32),
            pltpu.VMEM((_G, _D), jnp.float32),
            pltpu.VMEM((_G, _D), jnp.float32),
            pltpu.VMEM((_G, _D), jnp.float32),
            pltpu.VMEM((_G, _D), jnp.float32),
            pltpu.VMEM((_G, _D), jnp.float32),
            pltpu.VMEM((_G, _D), jnp.float32),
            pltpu.VMEM_SHARED((_NS, 16), jnp.int32),
            pltpu.SemaphoreType.DMA,
            pltpu.SemaphoreType.DMA,
            pltpu.SemaphoreType.DMA,
            pltpu.SemaphoreType.DMA,
            pltpu.SemaphoreType.DMA,
            pltpu.SemaphoreType.DMA,
            pltpu.SemaphoreType.DMA,
            pltpu.SemaphoreType.DMA,
            pltpu.SemaphoreType.DMA,
            pltpu.SemaphoreType.DMA,
            pltpu.SemaphoreType.DMA,
            pltpu.SemaphoreType.DMA,
        ],
    )(_body)
    return run(inp, weights).reshape(_BSZ, _SEQ, _D)


def kernel(input, weights):
    return lax.stop_gradient(_sc_embed(input, weights))


# speedup vs baseline: 1.0626x; 1.0008x over previous
import functools

import jax
import jax.numpy as jnp
from jax import lax
from jax.experimental import pallas as pl
from jax.experimental.pallas import tpu as pltpu
from jax.experimental.pallas import tpu_sc as plsc

_PAD = 1
_BSZ = 4
_SEQ = 8192
_D = 1024
_NC = 2
_NS = 16
_CHUNK = _SEQ // (_NC * _NS // _BSZ)
_CPB = _NC * _NS // _BSZ
_G = 32
_NSTEP = _CHUNK // _G
_NBUF = 3


def _body(inp_hbm, w_hbm, out_hbm, inp_v, idx_v, tot_tmp, totals_v,
          buf0, buf1, buf2, shared_totals,
          gsem0, gsem1, gsem2, wsem0, wsem1, wsem2):
    core = lax.axis_index("c")
    sid = lax.axis_index("s")
    b = core * (_NS // _CPB) + sid // _CPB
    chunk = sid % _CPB
    base = pl.multiple_of(chunk * _CHUNK, _CHUNK)

    pltpu.sync_copy(inp_hbm.at[b, pl.ds(base, _CHUNK)], inp_v)

    run = jnp.int32(0)
    for i in range(_CHUNK // 16):
        x = inp_v[pl.ds(i * 16, 16)]
        m = jnp.minimum(jnp.abs(x - _PAD), 1)
        cum = plsc.cumsum(m) + run
        idx_v[pl.ds(i * 16, 16)] = cum * m + _PAD
        run = run + jnp.sum(m)

    tot_tmp[...] = jnp.full((16,), run, jnp.int32)
    pltpu.sync_copy(tot_tmp, shared_totals.at[sid])
    plsc.subcore_barrier()
    pltpu.sync_copy(shared_totals, totals_v)
    row_base = (sid // _CPB) * _CPB
    off = jnp.int32(0)
    for j in range(_NS):
        tj = totals_v[j][0]
        take = jnp.logical_and(j >= row_base, j < sid)
        off = off + jnp.where(take, tj, jnp.int32(0))

    for i in range(_CHUNK // 16):
        v = idx_v[pl.ds(i * 16, 16)]
        mm = jnp.minimum(jnp.abs(v - _PAD), 1)
        idx_v[pl.ds(i * 16, 16)] = v + off * mm

    bufs = (buf0, buf1, buf2)
    gsems = (gsem0, gsem1, gsem2)
    wsems = (wsem0, wsem1, wsem2)

    def fire_gather(g):
        return pltpu.async_copy(
            w_hbm.at[idx_v.at[pl.ds(g * _G, _G)]],
            bufs[g % _NBUF], gsems[g % _NBUF])

    def fire_write(g):
        row = b * _SEQ + base + g * _G
        return pltpu.async_copy(
            bufs[g % _NBUF],
            out_hbm.at[pl.ds(row, _G), :], wsems[g % _NBUF])

    gh = [None] * _NSTEP
    wh = [None] * _NSTEP
    gh[0] = fire_gather(0)
    gh[1] = fire_gather(1)
    for g in range(_NSTEP):
        gh[g].wait()
        wh[g] = fire_write(g)
        h = g + _NBUF - 1
        if h < _NSTEP:
            if g >= 1:
                wh[g - 1].wait()
            gh[h] = fire_gather(h)
    wh[_NSTEP - 3].wait()
    wh[_NSTEP - 2].wait()
    wh[_NSTEP - 1].wait()


@jax.jit
def _sc_embed(inp, weights):
    mesh = plsc.VectorSubcoreMesh(core_axis_name="c", subcore_axis_name="s")
    run = functools.partial(
        pl.kernel,
        mesh=mesh,
        compiler_params=pltpu.CompilerParams(needs_layout_passes=False),
        out_type=jax.ShapeDtypeStruct((_BSZ * _SEQ, _D), jnp.float32),
        scratch_types=[
            pltpu.VMEM((_CHUNK,), jnp.int32),
            pltpu.VMEM((_CHUNK,), jnp.int32),
            pltpu.VMEM((16,), jnp.int32),
            pltpu.VMEM((_NS, 16), jnp.int32),
            pltpu.VMEM((_G, _D), jnp.float32),
            pltpu.VMEM((_G, _D), jnp.float32),
            pltpu.VMEM((_G, _D), jnp.float32),
            pltpu.VMEM_SHARED((_NS, 16), jnp.int32),
            pltpu.SemaphoreType.DMA,
            pltpu.SemaphoreType.DMA,
            pltpu.SemaphoreType.DMA,
            pltpu.SemaphoreType.DMA,
            pltpu.SemaphoreType.DMA,
            pltpu.SemaphoreType.DMA,
        ],
    )(_body)
    return run(inp, weights).reshape(_BSZ, _SEQ, _D)


def kernel(input, weights):
    return lax.stop_gradient(_sc_embed(input, weights))
